# baseline (speedup 1.0000x reference)
import functools

import jax
import jax.numpy as jnp
from jax import lax
from jax.experimental import pallas as pl
from jax.experimental.pallas import tpu as pltpu
from jax.experimental.pallas import tpu_sc as plsc

N_NODES = 10000
D_FEAT = 256
M_ROWS = 160000

TC_B = 256
TC_NB = 15
NT = TC_B * TC_NB
SC_N = N_NODES - NT

NUM_WORKERS = 32
NPW = 208
G = 8
RMAX = 144
LANES = 16
J = D_FEAT // LANES
ACT_PAD = N_NODES + 16

TC_CH = 544
TC_NCH = 8
TC_W = TC_CH * TC_NCH


def _hsum8(v, off):
  s = v[off]
  for i in range(1, 8):
    s = s + v[off + i]
  return s


def _row_window(row_off, gs):
  a = (row_off // 8) * 8
  need = row_off + gs - a
  nds = (need + 7) // 8
  dstart = jnp.minimum(a, M_ROWS - nds * 8)
  delta = row_off - dstart
  return dstart, delta, nds


def _sc_body(act_hbm, src_hbm, nbr_hbm, out_hbm, act_v,
             rows0, rows1, src0, src1, out0, out1,
             semr0, semr1, sems0, sems1, semo0, semo1):
  cid = lax.axis_index("c")
  sid = lax.axis_index("s")
  wid = sid * 2 + cid
  base = NT + wid * NPW
  n_nodes = jnp.maximum(jnp.minimum(NPW, N_NODES - base), 0)
  n_pairs = n_nodes // (2 * G)
  active = n_pairs > 0

  pltpu.sync_copy(act_hbm, act_v.at[pl.ds(0, N_NODES)])

  base_c = jnp.minimum(base, N_NODES)

  def _chunk_sum(i, acc):
    return acc + act_v[pl.ds(i * LANES, LANES)]

  acc0 = lax.fori_loop(0, base_c // LANES, _chunk_sum,
                       jnp.zeros((LANES,), jnp.int32))
  s = acc0[0]
  for i in range(1, LANES):
    s = s + acc0[i]
  row_off0 = base_c + s

  def _fire_rows(dstart, nds, rows_v, sem):
    def _f(d, c):
      pltpu.async_copy(nbr_hbm.at[pl.ds(dstart + d * 8, 8)],
                       rows_v.at[pl.ds(d * 8, 8)], sem)
      return c
    lax.fori_loop(0, nds, _f, 0)

  def _drain_rows(nds, rows_v, sem):
    def _f(d, c):
      pltpu.make_async_copy(nbr_hbm.at[pl.ds(0, 8)],
                            rows_v.at[pl.ds(0, 8)], sem).wait()
      return c
    lax.fori_loop(0, nds, _f, 0)

  def _compute_group(chunk, inv16, lane_off, delta, rows_v, src_v, out_v):
    pos = delta
    for i in range(G):
      cnt = chunk[lane_off + i] + 1
      inv = jnp.broadcast_to(inv16[lane_off + i], (LANES,))

      def _row_body(k, accs, pos=pos):
        r = pos + k
        return tuple(a + rows_v[r, pl.ds(j * LANES, LANES)]
                     for j, a in enumerate(accs))

      accs = lax.fori_loop(
          0, cnt, _row_body,
          tuple(jnp.zeros((LANES,), jnp.float32) for _ in range(J)))
      for j in range(J):
        h = (src_v[i, pl.ds(j * LANES, LANES)] + accs[j] * inv) * 0.5
        out_v[i, pl.ds(j * LANES, LANES)] = jnp.maximum(h, 0.0)
      pos = pos + cnt

  chunk0 = act_v[pl.ds(jnp.minimum(base, N_NODES - LANES), LANES)]
  gs0 = G + _hsum8(chunk0, 0)
  gs1 = G + _hsum8(chunk0, 8)
  ds0, d0, n0 = _row_window(row_off0, gs0)
  ds1, d1, n1 = _row_window(row_off0 + gs0, gs1)

  @pl.when(active)
  def _():
    _fire_rows(ds0, n0, rows0, semr0)
    pltpu.async_copy(src_hbm.at[pl.ds(base, G)], src0, sems0)
    _fire_rows(ds1, n1, rows1, semr1)
    pltpu.async_copy(src_hbm.at[pl.ds(base + G, G)], src1, sems1)

  def _pair_body(t, carry):
    chunk, d0, n0, d1, n1, row_off2 = carry
    cnt16 = chunk + 1
    inv16 = 1.0 / cnt16.astype(jnp.float32)
    chunk_n = act_v[pl.ds(jnp.minimum(base + (t + 1) * LANES, N_NODES),
                          LANES)]
    gs2 = G + _hsum8(chunk_n, 0)
    gs3 = G + _hsum8(chunk_n, 8)
    valid = t + 1 < n_pairs

    nbase0 = base + t * 2 * G
    pltpu.make_async_copy(src_hbm.at[pl.ds(0, G)], src0, sems0).wait()
    _drain_rows(n0, rows0, semr0)

    @pl.when(t > 0)
    def _():
      pltpu.make_async_copy(out0, out_hbm.at[pl.ds(0, G)], semo0).wait()

    _compute_group(chunk, inv16, 0, d0, rows0, src0, out0)
    pltpu.async_copy(out0, out_hbm.at[pl.ds(nbase0 - NT, G)], semo0)

    ds2, d2, n2 = _row_window(row_off2, gs2)

    @pl.when(valid)
    def _():
      _fire_rows(ds2, n2, rows0, semr0)
      pltpu.async_copy(src_hbm.at[pl.ds(nbase0 + 2 * G, G)], src0, sems0)

    nbase1 = nbase0 + G
    pltpu.make_async_copy(src_hbm.at[pl.ds(0, G)], src1, sems1).wait()
    _drain_rows(n1, rows1, semr1)

    @pl.when(t > 0)
    def _():
      pltpu.make_async_copy(out1, out_hbm.at[pl.ds(0, G)], semo1).wait()

    _compute_group(chunk, inv16, 8, d1, rows1, src1, out1)
    pltpu.async_copy(out1, out_hbm.at[pl.ds(nbase1 - NT, G)], semo1)

    row_off3 = row_off2 + gs2
    ds3, d3, n3 = _row_window(row_off3, gs3)

    @pl.when(valid)
    def _():
      _fire_rows(ds3, n3, rows1, semr1)
      pltpu.async_copy(src_hbm.at[pl.ds(nbase1 + 2 * G, G)], src1, sems1)

    return (chunk_n, d2, n2, d3, n3, row_off3 + gs3)

  lax.fori_loop(0, n_pairs, _pair_body,
                (chunk0, d0, n0, d1, n1, row_off0 + gs0 + gs1))

  @pl.when(active)
  def _():
    pltpu.make_async_copy(out0, out_hbm.at[pl.ds(0, G)], semo0).wait()
    pltpu.make_async_copy(out1, out_hbm.at[pl.ds(0, G)], semo1).wait()


def _sc_call(action, src, nbr):
  mesh = plsc.VectorSubcoreMesh(core_axis_name="c", subcore_axis_name="s")
  run = functools.partial(
      pl.kernel,
      out_type=jax.ShapeDtypeStruct((SC_N, D_FEAT), jnp.float32),
      mesh=mesh,
      scratch_types=[
          pltpu.VMEM((ACT_PAD,), jnp.int32),
          pltpu.VMEM((RMAX, D_FEAT), jnp.float32),
          pltpu.VMEM((RMAX, D_FEAT), jnp.float32),
          pltpu.VMEM((G, D_FEAT), jnp.float32),
          pltpu.VMEM((G, D_FEAT), jnp.float32),
          pltpu.VMEM((G, D_FEAT), jnp.float32),
          pltpu.VMEM((G, D_FEAT), jnp.float32),
          pltpu.SemaphoreType.DMA,
          pltpu.SemaphoreType.DMA,
          pltpu.SemaphoreType.DMA,
          pltpu.SemaphoreType.DMA,
          pltpu.SemaphoreType.DMA,
          pltpu.SemaphoreType.DMA,
      ],
  )(_sc_body)
  return run(action, src, nbr)


def _tc_body(act_ref, actn_ref, lt_ref, src_ref, nbr_hbm, out_ref,
             rows_v, sem, wref):
  b = pl.program_id(0)

  @pl.when(b == 0)
  def _():
    wref[0] = 0

  w0 = wref[0]
  cnt_col = act_ref[...] + 1
  cnt_f = cnt_col.astype(jnp.float32)
  needed = jnp.sum(cnt_col)
  wref[0] = w0 + needed

  ends = jax.lax.dot_general(lt_ref[...], cnt_f, (((1,), (0,)), ((), ())),
                             preferred_element_type=jnp.float32)
  starts = ends - cnt_f

  def _window(w0_, needed_):
    a_ = jnp.minimum((w0_ // 8) * 8, M_ROWS - TC_W)
    delta_ = w0_ - a_
    nch_ = (delta_ + needed_ + TC_CH - 1) // TC_CH
    return a_, delta_, nch_

  a0, delta, nch = _window(w0, needed)

  po = (b % 2) * TC_W

  def _fire(a_, nch_, po_):
    def _f(c, acc):
      pltpu.make_async_copy(nbr_hbm.at[pl.ds(a_ + c * TC_CH, TC_CH)],
                            rows_v.at[pl.ds(po_ + c * TC_CH, TC_CH)],
                            sem).start()
      return acc
    lax.fori_loop(0, nch_, _f, 0)

  @pl.when(b == 0)
  def _():
    _fire(a0, nch, 0)

  def _wait(c, acc):
    pltpu.make_async_copy(nbr_hbm.at[pl.ds(0, TC_CH)],
                          rows_v.at[pl.ds(0, TC_CH)], sem).wait()
    return acc
  lax.fori_loop(0, nch, _wait, 0)

  @pl.when(b + 1 < TC_NB)
  def _():
    a0n, _, nchn = _window(w0 + needed, jnp.sum(actn_ref[...]) + TC_B)
    _fire(a0n, nchn, (TC_W - po) % (2 * TC_W))

  st = starts.astype(jnp.int32) + delta
  en = ends.astype(jnp.int32) + delta
  ch_iota = lax.broadcasted_iota(jnp.int32, (TC_B, TC_CH), 1)

  def _chunk(c, acc):
    r_iota = ch_iota + c * TC_CH
    mask = ((r_iota >= st) & (r_iota < en)).astype(jnp.float32)
    rows = rows_v[pl.ds(po + c * TC_CH, TC_CH), :]
    return acc + jax.lax.dot_general(mask, rows, (((1,), (0,)), ((), ())),
                                     preferred_element_type=jnp.float32)

  sums = lax.fori_loop(0, nch, _chunk,
                       jnp.zeros((TC_B, D_FEAT), jnp.float32))
  inv = 1.0 / cnt_f
  out_ref[...] = jnp.maximum((src_ref[...] + sums * inv) * 0.5, 0.0)


def _tc_call(acts_col, lt, src, nbr):
  return pl.pallas_call(
      _tc_body,
      grid=(TC_NB,),
      in_specs=[
          pl.BlockSpec((TC_B, 1), lambda b: (b, 0)),
          pl.BlockSpec((TC_B, 1),
                       lambda b: (jnp.minimum(b + 1, TC_NB - 1), 0)),
          pl.BlockSpec((TC_B, TC_B), lambda b: (0, 0)),
          pl.BlockSpec((TC_B, D_FEAT), lambda b: (b, 0)),
          pl.BlockSpec(memory_space=pl.ANY),
      ],
      out_specs=pl.BlockSpec((TC_B, D_FEAT), lambda b: (b, 0)),
      out_shape=jax.ShapeDtypeStruct((NT, D_FEAT), jnp.float32),
      scratch_shapes=[
          pltpu.VMEM((2 * TC_W, D_FEAT), jnp.float32),
          pltpu.SemaphoreType.DMA,
          pltpu.SMEM((1,), jnp.int32),
      ],
  )(acts_col, acts_col, lt, src, nbr)


@jax.jit
def kernel(action, src_node_features, neighbor_node_features):
  sc_out = _sc_call(action, src_node_features, neighbor_node_features)
  acts_col = action[:NT, None]
  lt = jnp.tril(jnp.ones((TC_B, TC_B), jnp.float32))
  tc_out = _tc_call(acts_col, lt, src_node_features, neighbor_node_features)
  return jnp.concatenate([tc_out, sc_out], axis=0)


# speedup vs baseline: 1.0528x; 1.0528x over previous
import functools

import numpy as np

import jax
import jax.numpy as jnp
from jax import lax
from jax.experimental import pallas as pl
from jax.experimental.pallas import tpu as pltpu
from jax.experimental.pallas import tpu_sc as plsc

N_NODES = 10000
D_FEAT = 256
M_ROWS = 160000

TC_B = 256
TC_NB = 18
NT = TC_B * TC_NB
SC_N = N_NODES - NT

NUM_WORKERS = 32
NPW = 176
G = 8
RMAX = 144
LANES = 16
J = D_FEAT // LANES
ACT_PAD = N_NODES + 16

TC_CH = 544
TC_NCH = 8
TC_W = TC_CH * TC_NCH


def _hsum8(v, off):
  s = v[off]
  for i in range(1, 8):
    s = s + v[off + i]
  return s


def _row_window(row_off, gs):
  a = (row_off // 8) * 8
  need = row_off + gs - a
  nds = (need + 7) // 8
  dstart = jnp.minimum(a, M_ROWS - nds * 8)
  delta = row_off - dstart
  return dstart, delta, nds


def _sc_body(act_hbm, src_hbm, nbr_hbm, out_hbm, act_v,
             rows0, rows1, src0, src1, out0, out1,
             semr0, semr1, sems0, sems1, semo0, semo1):
  cid = lax.axis_index("c")
  sid = lax.axis_index("s")
  wid = sid * 2 + cid
  base = NT + wid * NPW
  n_nodes = jnp.maximum(jnp.minimum(NPW, N_NODES - base), 0)
  n_pairs = n_nodes // (2 * G)
  active = n_pairs > 0

  pltpu.sync_copy(act_hbm, act_v.at[pl.ds(0, N_NODES)])

  base_c = jnp.minimum(base, N_NODES)

  def _chunk_sum(i, acc):
    return acc + act_v[pl.ds(i * LANES, LANES)]

  acc0 = lax.fori_loop(0, base_c // LANES, _chunk_sum,
                       jnp.zeros((LANES,), jnp.int32))
  s = acc0[0]
  for i in range(1, LANES):
    s = s + acc0[i]
  row_off0 = base_c + s

  def _fire_rows(dstart, nds, rows_v, sem):
    def _f(d, c):
      pltpu.async_copy(nbr_hbm.at[pl.ds(dstart + d * 8, 8)],
                       rows_v.at[pl.ds(d * 8, 8)], sem)
      return c
    lax.fori_loop(0, nds, _f, 0)

  def _drain_rows(nds, rows_v, sem):
    def _f(d, c):
      pltpu.make_async_copy(nbr_hbm.at[pl.ds(0, 8)],
                            rows_v.at[pl.ds(0, 8)], sem).wait()
      return c
    lax.fori_loop(0, nds, _f, 0)

  def _compute_group(chunk, inv16, lane_off, delta, rows_v, src_v, out_v):
    pos = delta
    for i in range(G):
      cnt = chunk[lane_off + i] + 1
      inv = jnp.broadcast_to(inv16[lane_off + i], (LANES,))

      def _row_body(k, accs, pos=pos):
        r = pos + k
        return tuple(a + rows_v[r, pl.ds(j * LANES, LANES)]
                     for j, a in enumerate(accs))

      accs = lax.fori_loop(
          0, cnt, _row_body,
          tuple(jnp.zeros((LANES,), jnp.float32) for _ in range(J)))
      for j in range(J):
        h = (src_v[i, pl.ds(j * LANES, LANES)] + accs[j] * inv) * 0.5
        out_v[i, pl.ds(j * LANES, LANES)] = jnp.maximum(h, 0.0)
      pos = pos + cnt

  chunk0 = act_v[pl.ds(jnp.minimum(base, N_NODES - LANES), LANES)]
  gs0 = G + _hsum8(chunk0, 0)
  gs1 = G + _hsum8(chunk0, 8)
  ds0, d0, n0 = _row_window(row_off0, gs0)
  ds1, d1, n1 = _row_window(row_off0 + gs0, gs1)

  @pl.when(active)
  def _():
    _fire_rows(ds0, n0, rows0, semr0)
    pltpu.async_copy(src_hbm.at[pl.ds(base, G)], src0, sems0)
    _fire_rows(ds1, n1, rows1, semr1)
    pltpu.async_copy(src_hbm.at[pl.ds(base + G, G)], src1, sems1)

  def _pair_body(t, carry):
    chunk, d0, n0, d1, n1, row_off2 = carry
    cnt16 = chunk + 1
    inv16 = 1.0 / cnt16.astype(jnp.float32)
    chunk_n = act_v[pl.ds(jnp.minimum(base + (t + 1) * LANES, N_NODES),
                          LANES)]
    gs2 = G + _hsum8(chunk_n, 0)
    gs3 = G + _hsum8(chunk_n, 8)
    valid = t + 1 < n_pairs

    nbase0 = base + t * 2 * G
    pltpu.make_async_copy(src_hbm.at[pl.ds(0, G)], src0, sems0).wait()
    _drain_rows(n0, rows0, semr0)

    @pl.when(t > 0)
    def _():
      pltpu.make_async_copy(out0, out_hbm.at[pl.ds(0, G)], semo0).wait()

    _compute_group(chunk, inv16, 0, d0, rows0, src0, out0)
    pltpu.async_copy(out0, out_hbm.at[pl.ds(nbase0 - NT, G)], semo0)

    ds2, d2, n2 = _row_window(row_off2, gs2)

    @pl.when(valid)
    def _():
      _fire_rows(ds2, n2, rows0, semr0)
      pltpu.async_copy(src_hbm.at[pl.ds(nbase0 + 2 * G, G)], src0, sems0)

    nbase1 = nbase0 + G
    pltpu.make_async_copy(src_hbm.at[pl.ds(0, G)], src1, sems1).wait()
    _drain_rows(n1, rows1, semr1)

    @pl.when(t > 0)
    def _():
      pltpu.make_async_copy(out1, out_hbm.at[pl.ds(0, G)], semo1).wait()

    _compute_group(chunk, inv16, 8, d1, rows1, src1, out1)
    pltpu.async_copy(out1, out_hbm.at[pl.ds(nbase1 - NT, G)], semo1)

    row_off3 = row_off2 + gs2
    ds3, d3, n3 = _row_window(row_off3, gs3)

    @pl.when(valid)
    def _():
      _fire_rows(ds3, n3, rows1, semr1)
      pltpu.async_copy(src_hbm.at[pl.ds(nbase1 + 2 * G, G)], src1, sems1)

    return (chunk_n, d2, n2, d3, n3, row_off3 + gs3)

  lax.fori_loop(0, n_pairs, _pair_body,
                (chunk0, d0, n0, d1, n1, row_off0 + gs0 + gs1))

  @pl.when(active)
  def _():
    pltpu.make_async_copy(out0, out_hbm.at[pl.ds(0, G)], semo0).wait()
    pltpu.make_async_copy(out1, out_hbm.at[pl.ds(0, G)], semo1).wait()


def _sc_call(action, src, nbr):
  mesh = plsc.VectorSubcoreMesh(core_axis_name="c", subcore_axis_name="s")
  run = functools.partial(
      pl.kernel,
      out_type=jax.ShapeDtypeStruct((SC_N, D_FEAT), jnp.float32),
      mesh=mesh,
      scratch_types=[
          pltpu.VMEM((ACT_PAD,), jnp.int32),
          pltpu.VMEM((RMAX, D_FEAT), jnp.float32),
          pltpu.VMEM((RMAX, D_FEAT), jnp.float32),
          pltpu.VMEM((G, D_FEAT), jnp.float32),
          pltpu.VMEM((G, D_FEAT), jnp.float32),
          pltpu.VMEM((G, D_FEAT), jnp.float32),
          pltpu.VMEM((G, D_FEAT), jnp.float32),
          pltpu.SemaphoreType.DMA,
          pltpu.SemaphoreType.DMA,
          pltpu.SemaphoreType.DMA,
          pltpu.SemaphoreType.DMA,
          pltpu.SemaphoreType.DMA,
          pltpu.SemaphoreType.DMA,
      ],
  )(_sc_body)
  return run(action, src, nbr)


def _tc_body(act_ref, actn_ref, lt_ref, src_ref, nbr_hbm, out_ref,
             rows_v, sem, wref):
  b = pl.program_id(0)

  @pl.when(b == 0)
  def _():
    wref[0] = 0

  w0 = wref[0]
  cnt_col = act_ref[...] + 1
  cnt_f = cnt_col.astype(jnp.float32)
  needed = jnp.sum(cnt_col)
  wref[0] = w0 + needed

  ends = jax.lax.dot_general(lt_ref[...], cnt_f, (((1,), (0,)), ((), ())),
                             preferred_element_type=jnp.float32)
  starts = ends - cnt_f

  def _window(w0_, needed_):
    a_ = jnp.minimum((w0_ // 8) * 8, M_ROWS - TC_W)
    delta_ = w0_ - a_
    nch_ = (delta_ + needed_ + TC_CH - 1) // TC_CH
    return a_, delta_, nch_

  a0, delta, nch = _window(w0, needed)

  po = (b % 2) * TC_W

  def _fire(a_, nch_, po_):
    def _f(c, acc):
      pltpu.make_async_copy(nbr_hbm.at[pl.ds(a_ + c * TC_CH, TC_CH)],
                            rows_v.at[pl.ds(po_ + c * TC_CH, TC_CH)],
                            sem).start()
      return acc
    lax.fori_loop(0, nch_, _f, 0)

  @pl.when(b == 0)
  def _():
    _fire(a0, nch, 0)

  def _wait(c, acc):
    pltpu.make_async_copy(nbr_hbm.at[pl.ds(0, TC_CH)],
                          rows_v.at[pl.ds(0, TC_CH)], sem).wait()
    return acc
  lax.fori_loop(0, nch, _wait, 0)

  @pl.when(b + 1 < TC_NB)
  def _():
    a0n, _, nchn = _window(w0 + needed, jnp.sum(actn_ref[...]) + TC_B)
    _fire(a0n, nchn, (TC_W - po) % (2 * TC_W))

  st = starts.astype(jnp.int32) + delta
  en = ends.astype(jnp.int32) + delta
  ch_iota = lax.broadcasted_iota(jnp.int32, (TC_B, TC_CH), 1)

  def _chunk(c, acc):
    r_iota = ch_iota + c * TC_CH
    mask = ((r_iota >= st) & (r_iota < en)).astype(jnp.float32)
    rows = rows_v[pl.ds(po + c * TC_CH, TC_CH), :]
    return acc + jax.lax.dot_general(mask, rows, (((1,), (0,)), ((), ())),
                                     preferred_element_type=jnp.float32)

  sums = lax.fori_loop(0, nch, _chunk,
                       jnp.zeros((TC_B, D_FEAT), jnp.float32))
  inv = 1.0 / cnt_f
  out_ref[...] = jnp.maximum((src_ref[...] + sums * inv) * 0.5, 0.0)


def _tc_call(acts_col, lt, src, nbr):
  return pl.pallas_call(
      _tc_body,
      grid=(TC_NB,),
      in_specs=[
          pl.BlockSpec((TC_B, 1), lambda b: (b, 0)),
          pl.BlockSpec((TC_B, 1),
                       lambda b: (jnp.minimum(b + 1, TC_NB - 1), 0)),
          pl.BlockSpec((TC_B, TC_B), lambda b: (0, 0)),
          pl.BlockSpec((TC_B, D_FEAT), lambda b: (b, 0)),
          pl.BlockSpec(memory_space=pl.ANY),
      ],
      out_specs=pl.BlockSpec((TC_B, D_FEAT), lambda b: (b, 0)),
      out_shape=jax.ShapeDtypeStruct((NT, D_FEAT), jnp.float32),
      scratch_shapes=[
          pltpu.VMEM((2 * TC_W, D_FEAT), jnp.float32),
          pltpu.SemaphoreType.DMA,
          pltpu.SMEM((1,), jnp.int32),
      ],
  )(acts_col, acts_col, lt, src, nbr)


@jax.jit
def kernel(action, src_node_features, neighbor_node_features):
  sc_out = _sc_call(action, src_node_features, neighbor_node_features)
  acts_col = action[:NT, None]
  lt = jnp.asarray(np.tril(np.ones((TC_B, TC_B), np.float32)))
  tc_out = _tc_call(acts_col, lt, src_node_features, neighbor_node_features)
  out = jnp.zeros((N_NODES, D_FEAT), jnp.float32)
  out = lax.dynamic_update_slice(out, tc_out, (0, 0))
  return lax.dynamic_update_slice(out, sc_out, (NT, 0))
